# Initial kernel scaffold; baseline (speedup 1.0000x reference)
import jax
import jax.numpy as jnp
from jax.experimental import pallas as pl


def kernel(input, index):
    raise NotImplementedError("write your pallas kernel here")


# speedup vs baseline: 2.6765x; 2.6765x over previous
import functools

import jax
import jax.numpy as jnp
from jax import lax
from jax.experimental import pallas as pl
from jax.experimental.pallas import tpu as pltpu
from jax.experimental.pallas import tpu_sc as plsc

R = 128
C = 32768
NC = 2
NS = 16
NW = NC * NS
ROWS_PER_W = R // NW
L = 16
NVEC = C // L


def _gather_body(inp_hbm, idx_hbm, out_hbm, row_buf, idx_buf, out_buf):
    wid = lax.axis_index("s") * NC + lax.axis_index("c")
    for r in range(ROWS_PER_W):
        row = wid * ROWS_PER_W + r
        pltpu.sync_copy(inp_hbm.at[row], row_buf)
        pltpu.sync_copy(idx_hbm.at[row], idx_buf)

        def step(i, _):
            iv = idx_buf[pl.ds(i * L, L)]
            out_buf[pl.ds(i * L, L)] = plsc.load_gather(row_buf, [iv])
            return 0

        lax.fori_loop(0, NVEC, step, 0)
        pltpu.sync_copy(out_buf, out_hbm.at[row])


@functools.partial(jax.jit, donate_argnums=())
def _gather(inp, idx):
    k = functools.partial(
        pl.kernel,
        mesh=plsc.VectorSubcoreMesh(core_axis_name="c", subcore_axis_name="s"),
        out_type=jax.ShapeDtypeStruct((R, C), jnp.float32),
        scratch_types=[
            pltpu.VMEM((C,), jnp.float32),
            pltpu.VMEM((C,), jnp.int32),
            pltpu.VMEM((C,), jnp.float32),
        ],
        compiler_params=pltpu.CompilerParams(needs_layout_passes=False),
    )(_gather_body)
    return k(inp, idx)


def kernel(input, index):
    return _gather(input, index.astype(jnp.int32))


# speedup vs baseline: 4.6559x; 1.7396x over previous
import functools

import jax
import jax.numpy as jnp
from jax import lax
from jax.experimental import pallas as pl
from jax.experimental.pallas import tpu as pltpu
from jax.experimental.pallas import tpu_sc as plsc

R = 128
C = 32768
NC = 2
NS = 16
NW = NC * NS
ROWS_PER_W = R // NW
L = 16
NVEC = C // L


def _gather_body(inp_hbm, idx_hbm, out_hbm, row_buf, idx_buf, out_buf):
    wid = lax.axis_index("s") * NC + lax.axis_index("c")
    for r in range(ROWS_PER_W):
        row = wid * ROWS_PER_W + r
        pltpu.sync_copy(inp_hbm.at[row], row_buf)
        pltpu.sync_copy(idx_hbm.at[row], idx_buf)

        @plsc.parallel_loop(0, NVEC, unroll=8)
        def _(i):
            iv = idx_buf[pl.ds(i * L, L)]
            out_buf[pl.ds(i * L, L)] = plsc.load_gather(row_buf, [iv])
        pltpu.sync_copy(out_buf, out_hbm.at[row])


@functools.partial(jax.jit, donate_argnums=())
def _gather(inp, idx):
    k = functools.partial(
        pl.kernel,
        mesh=plsc.VectorSubcoreMesh(core_axis_name="c", subcore_axis_name="s"),
        out_type=jax.ShapeDtypeStruct((R, C), jnp.float32),
        scratch_types=[
            pltpu.VMEM((C,), jnp.float32),
            pltpu.VMEM((C,), jnp.int32),
            pltpu.VMEM((C,), jnp.float32),
        ],
        compiler_params=pltpu.CompilerParams(needs_layout_passes=False),
    )(_gather_body)
    return k(inp, idx)


def kernel(input, index):
    return _gather(input, index.astype(jnp.int32))
